# baseline (speedup 1.0000x reference)
import jax
import jax.numpy as jnp
from jax import lax
from jax.experimental import pallas as pl
from jax.experimental.pallas import tpu as pltpu
from jax.experimental.pallas import tpu_sc as plsc

N_TOK = 4096
D_IN = 2048
D_OUT = 2048
N_EXP = 8
TOPK = 2

N_ROWS = N_TOK * TOPK
TM = 256
NT = N_ROWS // TM
G = NT + N_EXP - 1
GP = 48
GTM = 256
GN = N_TOK // GTM

NC = 2
NS = 16
NW = NC * NS
L = 16


def _gate_body(x_ref, wg_ref, bg_ref, p0_ref, p1_ref, w0b_ref, w1b_ref,
               plan_ref, cnt_scr, idx_scr, rank_scr, w_scr):
    pid = pl.program_id(0)

    @pl.when(pid == 0)
    def _():
        cnt_scr[...] = jnp.zeros_like(cnt_scr)

    @pl.when(pid < GN)
    def _():
        g = jnp.tanh(
            jnp.dot(x_ref[...], wg_ref[...],
                    preferred_element_type=jnp.float32)
            + bg_ref[...])
        ei = lax.broadcasted_iota(jnp.int32, (GTM, N_EXP), 1)
        v0 = jnp.max(g, axis=1, keepdims=True)
        i0 = jnp.min(jnp.where(g == v0, ei, N_EXP), axis=1, keepdims=True)
        gm = jnp.where(ei == i0, -jnp.inf, g)
        v1 = jnp.max(gm, axis=1, keepdims=True)
        i1 = jnp.min(jnp.where(gm == v1, ei, N_EXP), axis=1, keepdims=True)
        z = jnp.exp(v1 - v0)
        w0 = 1.0 / (1.0 + z)
        w1 = z / (1.0 + z)

        oh = (ei == i0).astype(jnp.float32) + (ei == i1).astype(jnp.float32)
        rio = lax.broadcasted_iota(jnp.int32, (GTM, GTM), 0)
        cio = lax.broadcasted_iota(jnp.int32, (GTM, GTM), 1)
        tri = (rio > cio).astype(jnp.float32)
        strict = jnp.dot(tri, oh, preferred_element_type=jnp.float32)
        base = cnt_scr[...] + strict
        r0 = jnp.sum(jnp.where(ei == i0, base, 0.0), axis=1, keepdims=True)
        r1 = jnp.sum(jnp.where(ei == i1, base, 0.0), axis=1, keepdims=True)
        row = pid * GTM
        idx_scr[pl.ds(row, GTM), :] = jnp.concatenate([i0, i1], axis=1)
        rank_scr[pl.ds(row, GTM), :] = jnp.concatenate(
            [r0, r1], axis=1).astype(jnp.int32)
        w_scr[pl.ds(row, GTM), :] = jnp.concatenate([w0, w1], axis=1)
        cnt_scr[...] = cnt_scr[...] + jnp.sum(oh, axis=0, keepdims=True)

    @pl.when(pid == GN)
    def _():
        sizes_row = cnt_scr[...]
        e88r = lax.broadcasted_iota(jnp.int32, (N_EXP, N_EXP), 0)
        e88c = lax.broadcasted_iota(jnp.int32, (N_EXP, N_EXP), 1)
        eye = e88r == e88c
        sizes_col = jnp.sum(
            jnp.where(eye, jnp.broadcast_to(sizes_row, (N_EXP, N_EXP)), 0.0),
            axis=1, keepdims=True)
        a98r = lax.broadcasted_iota(jnp.int32, (N_EXP + 1, N_EXP), 0)
        a98c = lax.broadcasted_iota(jnp.int32, (N_EXP + 1, N_EXP), 1)
        offs_col = jnp.sum(
            jnp.where(a98c < a98r,
                      jnp.broadcast_to(sizes_row, (N_EXP + 1, N_EXP)), 0.0),
            axis=1, keepdims=True)
        offs_i = offs_col.astype(jnp.int32)
        szc = sizes_col.astype(jnp.int32)
        ts = offs_i[: N_EXP] // TM
        te = (offs_i[1:] - 1) // TM
        gt = jnp.where(szc > 0, te - ts + 1, 0)
        ss_row = jnp.sum(
            jnp.where(e88r < e88c, jnp.broadcast_to(gt, (N_EXP, N_EXP)), 0),
            axis=0, keepdims=True)
        ss = jnp.sum(
            jnp.where(eye, jnp.broadcast_to(ss_row, (N_EXP, N_EXP)), 0),
            axis=1, keepdims=True)
        total = jnp.sum(gt)
        s8g = lax.broadcasted_iota(jnp.int32, (N_EXP, GP), 1)
        e8g = lax.broadcasted_iota(jnp.int32, (N_EXP, GP), 0)
        started = (s8g >= ss).astype(jnp.int32)
        step_e = jnp.sum(started, axis=0, keepdims=True) - 1
        sel = e8g == step_e
        s1g = lax.broadcasted_iota(jnp.int32, (1, GP), 1)
        step_t = jnp.sum(jnp.where(sel, ts - ss, 0), axis=0,
                         keepdims=True) + s1g
        validv = s1g < total
        e_col = lax.broadcasted_iota(jnp.int32, (N_EXP, 1), 0)
        last_e = jnp.max(jnp.where(szc > 0, e_col, -1))
        step_e = jnp.where(validv, step_e, last_e)
        step_t = jnp.where(validv, step_t, NT - 1)
        prev_t = jnp.concatenate(
            [jnp.full((1, 1), -1, jnp.int32), step_t[:, : GP - 1]], axis=1)
        initv = ((step_t != prev_t) & validv).astype(jnp.int32)
        o9r = lax.broadcasted_iota(jnp.int32, (N_EXP + 1, GP), 0)
        o9c = lax.broadcasted_iota(jnp.int32, (N_EXP + 1, GP), 1)
        offs_row = jnp.sum(
            jnp.where(o9r == o9c,
                      jnp.broadcast_to(offs_i, (N_EXP + 1, GP)), 0),
            axis=0, keepdims=True)
        plan_ref[...] = jnp.concatenate(
            [step_e, step_t, validv.astype(jnp.int32), initv, offs_row],
            axis=0)

        idxa = idx_scr[...]
        ranka = rank_scr[...]
        eNr = lax.broadcasted_iota(jnp.int32, (N_TOK, N_EXP), 1)
        off8_row = jnp.sum(
            jnp.where(eye,
                      jnp.broadcast_to(offs_i[: N_EXP], (N_EXP, N_EXP)), 0),
            axis=0, keepdims=True)
        off8_b = jnp.broadcast_to(off8_row, (N_TOK, N_EXP))
        p0_ref[...] = ranka[:, 0:1] + jnp.sum(
            jnp.where(eNr == idxa[:, 0:1], off8_b, 0), axis=1, keepdims=True)
        p1_ref[...] = ranka[:, 1:2] + jnp.sum(
            jnp.where(eNr == idxa[:, 1:2], off8_b, 0), axis=1, keepdims=True)
        wa = w_scr[...]
        w0b_ref[...] = jnp.broadcast_to(wa[:, 0:1], (N_TOK, L))
        w1b_ref[...] = jnp.broadcast_to(wa[:, 1:2], (N_TOK, L))


def _gate(x, Wg, bg):
    return pl.pallas_call(
        _gate_body,
        grid=(GN + 1,),
        in_specs=[
            pl.BlockSpec((GTM, D_IN), lambda i: (jnp.minimum(i, GN - 1), 0)),
            pl.BlockSpec((D_IN, N_EXP), lambda i: (0, 0)),
            pl.BlockSpec((1, N_EXP), lambda i: (0, 0)),
        ],
        out_specs=[
            pl.BlockSpec((N_TOK, 1), lambda i: (0, 0)),
            pl.BlockSpec((N_TOK, 1), lambda i: (0, 0)),
            pl.BlockSpec((N_TOK, L), lambda i: (0, 0)),
            pl.BlockSpec((N_TOK, L), lambda i: (0, 0)),
            pl.BlockSpec((5, GP), lambda i: (0, 0)),
        ],
        out_shape=[
            jax.ShapeDtypeStruct((N_TOK, 1), jnp.int32),
            jax.ShapeDtypeStruct((N_TOK, 1), jnp.int32),
            jax.ShapeDtypeStruct((N_TOK, L), jnp.float32),
            jax.ShapeDtypeStruct((N_TOK, L), jnp.float32),
            jax.ShapeDtypeStruct((5, GP), jnp.int32),
        ],
        scratch_shapes=[
            pltpu.VMEM((1, N_EXP), jnp.float32),
            pltpu.VMEM((N_TOK, TOPK), jnp.int32),
            pltpu.VMEM((N_TOK, TOPK), jnp.int32),
            pltpu.VMEM((N_TOK, TOPK), jnp.float32),
        ],
        compiler_params=pltpu.CompilerParams(
            dimension_semantics=("arbitrary",)),
    )(x, Wg, bg.reshape(1, N_EXP))


_TPW = N_TOK // NW
_NCH = _TPW // L


def _sc_mesh():
    return plsc.VectorSubcoreMesh(
        core_axis_name="c", subcore_axis_name="s",
        num_cores=NC, num_subcores=NS)


def _sc_dispatch_body(x_hbm, p0_hbm, p1_hbm, xs_hbm, p0_v, p1_v, rows_v,
                      s0, s1):
    wid = lax.axis_index("s") * NC + lax.axis_index("c")
    base = wid * _TPW
    pltpu.sync_copy(p0_hbm.at[pl.ds(base, _TPW)], p0_v)
    pltpu.sync_copy(p1_hbm.at[pl.ds(base, _TPW)], p1_v)
    for c in range(_NCH):
        iv0 = p0_v[pl.ds(c * L, L)]
        iv1 = p1_v[pl.ds(c * L, L)]
        pltpu.sync_copy(x_hbm.at[pl.ds(base + c * L, L)], rows_v)
        ca = pltpu.async_copy(rows_v, xs_hbm.at[iv0], s0)
        cb = pltpu.async_copy(rows_v, xs_hbm.at[iv1], s1)
        ca.wait()
        cb.wait()


def _sc_dispatch(x, p0, p1):
    fn = pl.kernel(
        _sc_dispatch_body,
        mesh=_sc_mesh(),
        out_type=jax.ShapeDtypeStruct((N_ROWS, D_IN), jnp.float32),
        scratch_types=[
            pltpu.VMEM((_TPW,), jnp.int32),
            pltpu.VMEM((_TPW,), jnp.int32),
            pltpu.VMEM((L, D_IN), jnp.float32),
            pltpu.SemaphoreType.DMA,
            pltpu.SemaphoreType.DMA,
        ],
    )
    return fn(x, p0, p1)


def _moe_body(plan_ref, xs_ref, we_ref, be_ref, ga_ref, bt_ref, zs_ref):
    s = pl.program_id(0)
    e = plan_ref[0, s]
    t = plan_ref[1, s]
    valid = plan_ref[2, s]
    init = plan_ref[3, s]

    @pl.when(valid == 1)
    def _():
        h = jnp.dot(xs_ref[...], we_ref[0],
                    preferred_element_type=jnp.float32)
        h = h + be_ref[0]
        h = 0.5 * h * (1.0 + lax.erf(h / jnp.sqrt(2.0).astype(jnp.float32)))
        mu = jnp.mean(h, axis=1, keepdims=True)
        d = h - mu
        var = jnp.mean(d * d, axis=1, keepdims=True)
        hn = d / jnp.sqrt(var + 1e-5)
        val = hn * ga_ref[0] + bt_ref[0]
        r = t * TM + lax.broadcasted_iota(jnp.int32, (TM, 1), 0)
        mask = (r >= plan_ref[4, e]) & (r < plan_ref[4, e + 1])
        prev = jnp.where(init == 1, jnp.zeros_like(val), zs_ref[...])
        zs_ref[...] = jnp.where(mask, val, prev)


def _moe_matmul(plan, xs, We, be, gamma, beta):
    grid_spec = pltpu.PrefetchScalarGridSpec(
        num_scalar_prefetch=1,
        grid=(G,),
        in_specs=[
            pl.BlockSpec((TM, D_IN), lambda s, pn: (pn[1, s], 0)),
            pl.BlockSpec((1, D_IN, D_OUT), lambda s, pn: (pn[0, s], 0, 0)),
            pl.BlockSpec((1, 1, D_OUT), lambda s, pn: (pn[0, s], 0, 0)),
            pl.BlockSpec((1, 1, D_OUT), lambda s, pn: (pn[0, s], 0, 0)),
            pl.BlockSpec((1, 1, D_OUT), lambda s, pn: (pn[0, s], 0, 0)),
        ],
        out_specs=pl.BlockSpec((TM, D_OUT), lambda s, pn: (pn[1, s], 0)),
    )
    return pl.pallas_call(
        _moe_body,
        grid_spec=grid_spec,
        out_shape=jax.ShapeDtypeStruct((N_ROWS, D_OUT), jnp.float32),
        compiler_params=pltpu.CompilerParams(
            dimension_semantics=("arbitrary",)),
    )(plan, xs, We,
      be.reshape(N_EXP, 1, D_OUT), gamma.reshape(N_EXP, 1, D_OUT),
      beta.reshape(N_EXP, 1, D_OUT))


def _sc_combine_body(zs_hbm, p0_hbm, p1_hbm, w0_hbm, w1_hbm, out_hbm,
                     p0_v, p1_v, w0_v, w1_v, a_v, b_v, s0, s1):
    wid = lax.axis_index("s") * NC + lax.axis_index("c")
    base = wid * _TPW
    pltpu.sync_copy(p0_hbm.at[pl.ds(base, _TPW)], p0_v)
    pltpu.sync_copy(p1_hbm.at[pl.ds(base, _TPW)], p1_v)
    pltpu.sync_copy(w0_hbm.at[pl.ds(base, _TPW)], w0_v)
    pltpu.sync_copy(w1_hbm.at[pl.ds(base, _TPW)], w1_v)
    for c in range(_NCH):
        iv0 = p0_v[pl.ds(c * L, L)]
        iv1 = p1_v[pl.ds(c * L, L)]
        ca = pltpu.async_copy(zs_hbm.at[iv0], a_v, s0)
        cb = pltpu.async_copy(zs_hbm.at[iv1], b_v, s1)
        ca.wait()
        cb.wait()

        def row_body(i, carry):
            w0s = w0_v[c * L + i, pl.ds(0, L)]
            w1s = w1_v[c * L + i, pl.ds(0, L)]
            for j in range(D_OUT // L):
                a = a_v[i, pl.ds(j * L, L)]
                b = b_v[i, pl.ds(j * L, L)]
                a_v[i, pl.ds(j * L, L)] = w0s * a + w1s * b
            return carry

        lax.fori_loop(0, L, row_body, 0)
        pltpu.sync_copy(a_v, out_hbm.at[pl.ds(base + c * L, L)])


def _sc_combine(zs, p0, p1, w0b, w1b):
    fn = pl.kernel(
        _sc_combine_body,
        mesh=_sc_mesh(),
        out_type=jax.ShapeDtypeStruct((N_TOK, D_OUT), jnp.float32),
        scratch_types=[
            pltpu.VMEM((_TPW,), jnp.int32),
            pltpu.VMEM((_TPW,), jnp.int32),
            pltpu.VMEM((_TPW, L), jnp.float32),
            pltpu.VMEM((_TPW, L), jnp.float32),
            pltpu.VMEM((L, D_OUT), jnp.float32),
            pltpu.VMEM((L, D_OUT), jnp.float32),
            pltpu.SemaphoreType.DMA,
            pltpu.SemaphoreType.DMA,
        ],
    )
    return fn(zs, p0, p1, w0b, w1b)


def kernel(x, Wg, bg, We, be, gamma, beta):
    p0, p1, w0b, w1b, plan = _gate(x, Wg, bg)
    p0f = p0.reshape(-1)
    p1f = p1.reshape(-1)
    xs = _sc_dispatch(x, p0f, p1f)
    zs = _moe_matmul(plan, xs, We, be, gamma, beta)
    return _sc_combine(zs, p0f, p1f, w0b, w1b)


# speedup vs baseline: 1.0303x; 1.0303x over previous
import jax
import jax.numpy as jnp
from jax import lax
from jax.experimental import pallas as pl
from jax.experimental.pallas import tpu as pltpu
from jax.experimental.pallas import tpu_sc as plsc

N_TOK = 4096
D_IN = 2048
D_OUT = 2048
N_EXP = 8
TOPK = 2

N_ROWS = N_TOK * TOPK
TM = 256
NT = N_ROWS // TM
G = NT + N_EXP - 1
GP = 48
GTM = 256
GN = N_TOK // GTM

NC = 2
NS = 16
NW = NC * NS
L = 16


def _gate_body(x_ref, wg_ref, bg_ref, p0_ref, p1_ref, w0b_ref,
               w1b_ref, plan_ref, cnt_scr, idx_scr, rank_scr, w_scr):
    pid = pl.program_id(0)

    @pl.when(pid == 0)
    def _():
        cnt_scr[...] = jnp.zeros_like(cnt_scr)

    @pl.when(pid < GN)
    def _():
        g = jnp.tanh(
            jnp.dot(x_ref[...], wg_ref[...],
                    preferred_element_type=jnp.float32)
            + bg_ref[...])
        ei = lax.broadcasted_iota(jnp.int32, (GTM, N_EXP), 1)
        v0 = jnp.max(g, axis=1, keepdims=True)
        i0 = jnp.min(jnp.where(g == v0, ei, N_EXP), axis=1, keepdims=True)
        gm = jnp.where(ei == i0, -jnp.inf, g)
        v1 = jnp.max(gm, axis=1, keepdims=True)
        i1 = jnp.min(jnp.where(gm == v1, ei, N_EXP), axis=1, keepdims=True)
        z = jnp.exp(v1 - v0)
        w0 = 1.0 / (1.0 + z)
        w1 = z / (1.0 + z)

        oh = (ei == i0).astype(jnp.float32) + (ei == i1).astype(jnp.float32)
        rio = lax.broadcasted_iota(jnp.int32, (GTM, GTM), 0)
        cio = lax.broadcasted_iota(jnp.int32, (GTM, GTM), 1)
        tri = (rio > cio).astype(jnp.float32)
        strict = jnp.dot(tri, oh, preferred_element_type=jnp.float32)
        base = cnt_scr[...] + strict
        r0 = jnp.sum(jnp.where(ei == i0, base, 0.0), axis=1, keepdims=True)
        r1 = jnp.sum(jnp.where(ei == i1, base, 0.0), axis=1, keepdims=True)
        row = pid * GTM
        idx_scr[pl.ds(row, GTM), :] = jnp.concatenate([i0, i1], axis=1)
        rank_scr[pl.ds(row, GTM), :] = jnp.concatenate(
            [r0, r1], axis=1).astype(jnp.int32)
        w_scr[pl.ds(row, GTM), :] = jnp.concatenate([w0, w1], axis=1)
        cnt_scr[...] = cnt_scr[...] + jnp.sum(oh, axis=0, keepdims=True)

    @pl.when(pid == GN)
    def _():
        sizes_row = cnt_scr[...]
        e88r = lax.broadcasted_iota(jnp.int32, (N_EXP, N_EXP), 0)
        e88c = lax.broadcasted_iota(jnp.int32, (N_EXP, N_EXP), 1)
        eye = e88r == e88c
        sizes_col = jnp.sum(
            jnp.where(eye, jnp.broadcast_to(sizes_row, (N_EXP, N_EXP)), 0.0),
            axis=1, keepdims=True)
        a98r = lax.broadcasted_iota(jnp.int32, (N_EXP + 1, N_EXP), 0)
        a98c = lax.broadcasted_iota(jnp.int32, (N_EXP + 1, N_EXP), 1)
        offs_col = jnp.sum(
            jnp.where(a98c < a98r,
                      jnp.broadcast_to(sizes_row, (N_EXP + 1, N_EXP)), 0.0),
            axis=1, keepdims=True)
        offs_i = offs_col.astype(jnp.int32)
        szc = sizes_col.astype(jnp.int32)
        ts = offs_i[: N_EXP] // TM
        te = (offs_i[1:] - 1) // TM
        gt = jnp.where(szc > 0, te - ts + 1, 0)
        ss_row = jnp.sum(
            jnp.where(e88r < e88c, jnp.broadcast_to(gt, (N_EXP, N_EXP)), 0),
            axis=0, keepdims=True)
        ss = jnp.sum(
            jnp.where(eye, jnp.broadcast_to(ss_row, (N_EXP, N_EXP)), 0),
            axis=1, keepdims=True)
        total = jnp.sum(gt)
        s8g = lax.broadcasted_iota(jnp.int32, (N_EXP, GP), 1)
        e8g = lax.broadcasted_iota(jnp.int32, (N_EXP, GP), 0)
        started = (s8g >= ss).astype(jnp.int32)
        step_e = jnp.sum(started, axis=0, keepdims=True) - 1
        sel = e8g == step_e
        s1g = lax.broadcasted_iota(jnp.int32, (1, GP), 1)
        step_t = jnp.sum(jnp.where(sel, ts - ss, 0), axis=0,
                         keepdims=True) + s1g
        validv = s1g < total
        e_col = lax.broadcasted_iota(jnp.int32, (N_EXP, 1), 0)
        last_e = jnp.max(jnp.where(szc > 0, e_col, -1))
        step_e = jnp.where(validv, step_e, last_e)
        step_t = jnp.where(validv, step_t, NT - 1)
        prev_t = jnp.concatenate(
            [jnp.full((1, 1), -1, jnp.int32), step_t[:, : GP - 1]], axis=1)
        initv = ((step_t != prev_t) & validv).astype(jnp.int32)
        o9r = lax.broadcasted_iota(jnp.int32, (N_EXP + 1, GP), 0)
        o9c = lax.broadcasted_iota(jnp.int32, (N_EXP + 1, GP), 1)
        offs_row = jnp.sum(
            jnp.where(o9r == o9c,
                      jnp.broadcast_to(offs_i, (N_EXP + 1, GP)), 0),
            axis=0, keepdims=True)
        plan_ref[...] = jnp.concatenate(
            [step_e, step_t, validv.astype(jnp.int32), initv, offs_row],
            axis=0)

        idxa = idx_scr[...]
        ranka = rank_scr[...]
        eNr = lax.broadcasted_iota(jnp.int32, (N_TOK, N_EXP), 1)
        off8_row = jnp.sum(
            jnp.where(eye,
                      jnp.broadcast_to(offs_i[: N_EXP], (N_EXP, N_EXP)), 0),
            axis=0, keepdims=True)
        off8_b = jnp.broadcast_to(off8_row, (N_TOK, N_EXP))
        p0_ref[...] = ranka[:, 0:1] + jnp.sum(
            jnp.where(eNr == idxa[:, 0:1], off8_b, 0), axis=1, keepdims=True)
        p1_ref[...] = ranka[:, 1:2] + jnp.sum(
            jnp.where(eNr == idxa[:, 1:2], off8_b, 0), axis=1, keepdims=True)
        wa = w_scr[...]
        w0b_ref[...] = jnp.broadcast_to(wa[:, 0:1], (N_TOK, L))
        w1b_ref[...] = jnp.broadcast_to(wa[:, 1:2], (N_TOK, L))


def _gate(x, Wg, bg):
    return pl.pallas_call(
        _gate_body,
        grid=(GN + 1,),
        in_specs=[
            pl.BlockSpec((GTM, D_IN), lambda i: (jnp.minimum(i, GN - 1), 0)),
            pl.BlockSpec((D_IN, N_EXP), lambda i: (0, 0)),
            pl.BlockSpec((1, N_EXP), lambda i: (0, 0)),
        ],
        out_specs=[
            pl.BlockSpec((N_TOK, 1), lambda i: (0, 0)),
            pl.BlockSpec((N_TOK, 1), lambda i: (0, 0)),
            pl.BlockSpec((N_TOK, L), lambda i: (0, 0)),
            pl.BlockSpec((N_TOK, L), lambda i: (0, 0)),
            pl.BlockSpec((5, GP), lambda i: (0, 0)),
        ],
        out_shape=[
            jax.ShapeDtypeStruct((N_TOK, 1), jnp.int32),
            jax.ShapeDtypeStruct((N_TOK, 1), jnp.int32),
            jax.ShapeDtypeStruct((N_TOK, L), jnp.float32),
            jax.ShapeDtypeStruct((N_TOK, L), jnp.float32),
            jax.ShapeDtypeStruct((5, GP), jnp.int32),
        ],
        scratch_shapes=[
            pltpu.VMEM((1, N_EXP), jnp.float32),
            pltpu.VMEM((N_TOK, TOPK), jnp.int32),
            pltpu.VMEM((N_TOK, TOPK), jnp.int32),
            pltpu.VMEM((N_TOK, TOPK), jnp.float32),
        ],
        compiler_params=pltpu.CompilerParams(
            dimension_semantics=("arbitrary",)),
    )(x, Wg, bg.reshape(1, N_EXP))


_TPW = N_TOK // NW
_NCH = _TPW // L


def _sc_mesh():
    return plsc.VectorSubcoreMesh(
        core_axis_name="c", subcore_axis_name="s",
        num_cores=NC, num_subcores=NS)


def _sc_dispatch_body(x_hbm, p0_hbm, p1_hbm, xs_hbm, p0_v, p1_v,
                      rows0_v, rows1_v, sl0, sl1, s0, s1):
    wid = lax.axis_index("s") * NC + lax.axis_index("c")
    base = wid * _TPW
    pltpu.sync_copy(p0_hbm.at[pl.ds(base, _TPW)], p0_v)
    pltpu.sync_copy(p1_hbm.at[pl.ds(base, _TPW)], p1_v)
    rows = [rows0_v, rows1_v]
    sems = [sl0, sl1]
    ld = [None, None]
    ld[0] = pltpu.async_copy(x_hbm.at[pl.ds(base, L)], rows0_v, sl0)
    for c in range(_NCH):
        if c + 1 < _NCH:
            ld[(c + 1) % 2] = pltpu.async_copy(
                x_hbm.at[pl.ds(base + (c + 1) * L, L)],
                rows[(c + 1) % 2], sems[(c + 1) % 2])
        ld[c % 2].wait()
        iv0 = p0_v[pl.ds(c * L, L)]
        iv1 = p1_v[pl.ds(c * L, L)]
        ca = pltpu.async_copy(rows[c % 2], xs_hbm.at[iv0], s0)
        cb = pltpu.async_copy(rows[c % 2], xs_hbm.at[iv1], s1)
        ca.wait()
        cb.wait()


def _sc_dispatch(x, p0, p1):
    fn = pl.kernel(
        _sc_dispatch_body,
        mesh=_sc_mesh(),
        out_type=jax.ShapeDtypeStruct((N_ROWS, D_IN), jnp.float32),
        scratch_types=[
            pltpu.VMEM((_TPW,), jnp.int32),
            pltpu.VMEM((_TPW,), jnp.int32),
            pltpu.VMEM((L, D_IN), jnp.float32),
            pltpu.VMEM((L, D_IN), jnp.float32),
            pltpu.SemaphoreType.DMA,
            pltpu.SemaphoreType.DMA,
            pltpu.SemaphoreType.DMA,
            pltpu.SemaphoreType.DMA,
        ],
    )
    return fn(x, p0, p1)


def _moe_body(plan_ref, xs_ref, we_ref, be_ref, ga_ref, bt_ref, zs_ref):
    s = pl.program_id(0)
    e = plan_ref[0, s]
    t = plan_ref[1, s]
    valid = plan_ref[2, s]
    init = plan_ref[3, s]

    @pl.when(valid == 1)
    def _():
        h = jnp.dot(xs_ref[...], we_ref[0],
                    preferred_element_type=jnp.float32)
        h = h + be_ref[0]
        h = 0.5 * h * (1.0 + lax.erf(h / jnp.sqrt(2.0).astype(jnp.float32)))
        mu = jnp.mean(h, axis=1, keepdims=True)
        d = h - mu
        var = jnp.mean(d * d, axis=1, keepdims=True)
        hn = d / jnp.sqrt(var + 1e-5)
        val = hn * ga_ref[0] + bt_ref[0]
        r = t * TM + lax.broadcasted_iota(jnp.int32, (TM, 1), 0)
        mask = (r >= plan_ref[4, e]) & (r < plan_ref[4, e + 1])
        prev = jnp.where(init == 1, jnp.zeros_like(val), zs_ref[...])
        zs_ref[...] = jnp.where(mask, val, prev)


def _moe_matmul(plan, xs, We, be, gamma, beta):
    grid_spec = pltpu.PrefetchScalarGridSpec(
        num_scalar_prefetch=1,
        grid=(G,),
        in_specs=[
            pl.BlockSpec((TM, D_IN), lambda s, pn: (pn[1, s], 0)),
            pl.BlockSpec((1, D_IN, D_OUT), lambda s, pn: (pn[0, s], 0, 0)),
            pl.BlockSpec((1, 1, D_OUT), lambda s, pn: (pn[0, s], 0, 0)),
            pl.BlockSpec((1, 1, D_OUT), lambda s, pn: (pn[0, s], 0, 0)),
            pl.BlockSpec((1, 1, D_OUT), lambda s, pn: (pn[0, s], 0, 0)),
        ],
        out_specs=pl.BlockSpec((TM, D_OUT), lambda s, pn: (pn[1, s], 0)),
    )
    return pl.pallas_call(
        _moe_body,
        grid_spec=grid_spec,
        out_shape=jax.ShapeDtypeStruct((N_ROWS, D_OUT), jnp.float32),
        compiler_params=pltpu.CompilerParams(
            dimension_semantics=("arbitrary",)),
    )(plan, xs, We,
      be.reshape(N_EXP, 1, D_OUT), gamma.reshape(N_EXP, 1, D_OUT),
      beta.reshape(N_EXP, 1, D_OUT))


def _sc_combine_body(zs_hbm, p0_hbm, p1_hbm, w0_hbm, w1_hbm, out_hbm,
                     p0_v, p1_v, w0_v, w1_v, a0_v, a1_v, b_v,
                     sg0, sg1, sst):
    wid = lax.axis_index("s") * NC + lax.axis_index("c")
    base = wid * _TPW
    pltpu.sync_copy(p0_hbm.at[pl.ds(base, _TPW)], p0_v)
    pltpu.sync_copy(p1_hbm.at[pl.ds(base, _TPW)], p1_v)
    abufs = [a0_v, a1_v]
    st = [None, None]

    def gathers(c):
        iv0 = p0_v[pl.ds(c * L, L)]
        iv1 = p1_v[pl.ds(c * L, L)]
        return (pltpu.async_copy(zs_hbm.at[iv0], abufs[c % 2], sg0),
                pltpu.async_copy(zs_hbm.at[iv1], b_v, sg1))

    ga, gb = gathers(0)
    for c in range(_NCH):
        a_v = abufs[c % 2]
        pltpu.sync_copy(w0_hbm.at[pl.ds(base + c * L, L)], w0_v)
        pltpu.sync_copy(w1_hbm.at[pl.ds(base + c * L, L)], w1_v)
        ga.wait()
        gb.wait()

        def row_body(i, carry):
            w0s = w0_v[i, pl.ds(0, L)]
            w1s = w1_v[i, pl.ds(0, L)]
            for j in range(D_OUT // L):
                a = a_v[i, pl.ds(j * L, L)]
                b = b_v[i, pl.ds(j * L, L)]
                a_v[i, pl.ds(j * L, L)] = w0s * a + w1s * b
            return carry

        lax.fori_loop(0, L, row_body, 0)
        if c + 1 < _NCH:
            if st[(c + 1) % 2] is not None:
                st[(c + 1) % 2].wait()
                st[(c + 1) % 2] = None
            ga, gb = gathers(c + 1)
        st[c % 2] = pltpu.async_copy(
            a_v, out_hbm.at[pl.ds(base + c * L, L)], sst)
    for d in st:
        if d is not None:
            d.wait()


def _sc_combine(zs, p0, p1, w0b, w1b):
    fn = pl.kernel(
        _sc_combine_body,
        mesh=_sc_mesh(),
        out_type=jax.ShapeDtypeStruct((N_TOK, D_OUT), jnp.float32),
        scratch_types=[
            pltpu.VMEM((_TPW,), jnp.int32),
            pltpu.VMEM((_TPW,), jnp.int32),
            pltpu.VMEM((L, L), jnp.float32),
            pltpu.VMEM((L, L), jnp.float32),
            pltpu.VMEM((L, D_OUT), jnp.float32),
            pltpu.VMEM((L, D_OUT), jnp.float32),
            pltpu.VMEM((L, D_OUT), jnp.float32),
            pltpu.SemaphoreType.DMA,
            pltpu.SemaphoreType.DMA,
            pltpu.SemaphoreType.DMA,
        ],
    )
    return fn(zs, p0, p1, w0b, w1b)


def kernel(x, Wg, bg, We, be, gamma, beta):
    p0, p1, w0b, w1b, plan = _gate(x, Wg, bg)
    p0f = p0.reshape(-1)
    p1f = p1.reshape(-1)
    xs = _sc_dispatch(x, p0f, p1f)
    zs = _moe_matmul(plan, xs, We, be, gamma, beta)
    return _sc_combine(zs, p0f, p1f, w0b, w1b)
